# baseline (speedup 1.0000x reference)
import jax
import jax.numpy as jnp
from jax import lax
from jax.experimental import pallas as pl
from jax.experimental.pallas import tpu as pltpu
from jax.experimental.pallas import tpu_sc as plsc

N = 10000
D = 128
E = 320000

NC = 2
NS = 16
NW = NC * NS
C = 128
NCH = 80
ET = NCH * C
EP = NW * ET
NBUF = 3
NP = 10240
NPA = 10112
RPT = NP // NS
RPTA = NPA // NS
DC = 128
DNCH = ET // DC

_sc_mesh = plsc.VectorSubcoreMesh(core_axis_name="c", subcore_axis_name="s")


def _deg_body(dst_hbm, dp_hbm, dst_v, ones_v, zb_v, hist, ):
    cid = lax.axis_index("c")
    sid = lax.axis_index("s")
    wid = sid * NC + cid

    def fill_ones(i, _):
        ones_v[pl.ds(i * 16, 16)] = jnp.ones((16,), jnp.float32)
        return 0
    lax.fori_loop(0, DC // 16, fill_ones, 0)

    def fill_zero(i, _):
        zb_v[pl.ds(i * 16, 16)] = jnp.zeros((16,), jnp.float32)
        return 0
    lax.fori_loop(0, RPT // 16, fill_zero, 0)
    pltpu.sync_copy(zb_v, hist.at[pl.ds(sid * RPT, RPT)])
    plsc.subcore_barrier()

    pltpu.sync_copy(dst_hbm.at[wid], dst_v)

    def chunk(j, _):
        pltpu.sync_copy(ones_v, hist.at[dst_v.at[j]], add=True)
        return 0
    lax.fori_loop(0, DNCH, chunk, 0)

    plsc.subcore_barrier()
    pltpu.sync_copy(hist.at[pl.ds(sid * RPT, RPT)],
                    dp_hbm.at[cid, pl.ds(sid * RPT, RPT)])


_deg = pl.kernel(
    _deg_body,
    out_type=jax.ShapeDtypeStruct((NC, NP), jnp.float32),
    mesh=_sc_mesh,
    scratch_types=[
        pltpu.VMEM((DNCH, DC), jnp.int32),
        pltpu.VMEM((DC,), jnp.float32),
        pltpu.VMEM((RPT,), jnp.float32),
        pltpu.VMEM_SHARED((NP,), jnp.float32),
    ],
)


def _agg_body(y_hbm, src_hbm, dst_hbm, p_hbm, sbuf, dbuf, buf, acc, *sems):
    cid = lax.axis_index("c")
    sid = lax.axis_index("s")
    wid = sid * NC + cid
    sem_i = sems[0:NBUF]
    sem_d = sems[NBUF:2 * NBUF]
    sem_g = sems[2 * NBUF:3 * NBUF]
    sem_s = sems[3 * NBUF:4 * NBUF]
    G = NBUF - 1

    def fill_zero(i, _):
        r = i // (D // 16)
        c0 = (i % (D // 16)) * 16
        buf[0, r, pl.ds(c0, 16)] = jnp.zeros((16,), jnp.float32)
        return 0
    lax.fori_loop(0, C * (D // 16), fill_zero, 0)
    for k in range(RPTA // C):
        pltpu.sync_copy(buf.at[0], acc.at[pl.ds(sid * RPTA + k * C, C)])
    rem = RPTA % C
    if rem:
        pltpu.sync_copy(buf.at[0, pl.ds(0, rem)],
                        acc.at[pl.ds(sid * RPTA + (RPTA // C) * C, rem)])
    plsc.subcore_barrier()

    def idx_load(j, b):
        pltpu.async_copy(src_hbm.at[wid, j], sbuf.at[b], sem_i[b])

    def idx_wait(j, b):
        pltpu.make_async_copy(src_hbm.at[wid, j], sbuf.at[b], sem_i[b]).wait()

    def dst_load(j, b):
        pltpu.async_copy(dst_hbm.at[wid, j], dbuf.at[b], sem_d[b])

    def dst_wait(j, b):
        pltpu.make_async_copy(dst_hbm.at[wid, j], dbuf.at[b], sem_d[b]).wait()

    def gather_start(b):
        pltpu.async_copy(y_hbm.at[sbuf.at[b]], buf.at[b], sem_g[b])

    def gather_wait(b):
        pltpu.make_async_copy(y_hbm.at[sbuf.at[b]], buf.at[b], sem_g[b]).wait()

    def scat_start(b):
        pltpu.async_copy(buf.at[b], acc.at[dbuf.at[b]], sem_s[b], add=True)

    def scat_wait(b):
        pltpu.make_async_copy(buf.at[b], acc.at[dbuf.at[b]], sem_s[b]).wait()

    def slot(j, b, first=False):
        gather_wait(b)
        idx_load(j + NBUF, b)
        dst_wait(j, b)
        scat_start(b)
        if not first:
            scat_wait((b - 1) % NBUF)
        dst_load(j + NBUF - 1, (b - 1) % NBUF)
        idx_wait(j + G, (b + G) % NBUF)
        gather_start((b + G) % NBUF)

    for k in range(NBUF):
        idx_load(k, k)
    for k in range(NBUF - 1):
        dst_load(k, k)
    for k in range(G):
        idx_wait(k, k)
        gather_start(k)

    slot(0, 0, first=True)

    def ring(i, _):
        for b in range(NBUF):
            slot(NBUF * i + 1 + b, (1 + b) % NBUF)
        return 0
    K = (NCH - 1) // NBUF
    lax.fori_loop(0, K, ring, 0)
    for j in range(NBUF * K + 1, NCH):
        slot(j, j % NBUF)

    scat_wait((NCH - 1) % NBUF)
    for j in range(NCH, NCH + G):
        gather_wait(j % NBUF)
    idx_wait(NCH + NBUF - 1, (NCH + NBUF - 1) % NBUF)
    for j in range(NCH, NCH + NBUF - 1):
        dst_wait(j, j % NBUF)

    plsc.subcore_barrier()
    pltpu.sync_copy(acc.at[pl.ds(sid * RPTA, RPTA)],
                    p_hbm.at[cid, pl.ds(sid * RPTA, RPTA)])


_agg = pl.kernel(
    _agg_body,
    out_type=jax.ShapeDtypeStruct((NC, NP, D), jnp.float32),
    mesh=_sc_mesh,
    scratch_types=[
        pltpu.VMEM((NBUF, C), jnp.int32),
        pltpu.VMEM((NBUF, C), jnp.int32),
        pltpu.VMEM((NBUF, C, D), jnp.float32),
        pltpu.VMEM_SHARED((NPA, D), jnp.float32),
    ] + [pltpu.SemaphoreType.DMA] * (4 * NBUF),
)

BR = 256
GR = NP // BR


def _dinv_of(dg):
    return lax.rsqrt(dg[:, 0:1] + dg[:, 1:2] + 1.0)


def _mm1_body(x_ref, w_ref, dg_ref, y_ref):
    dinv = _dinv_of(dg_ref[...])
    xw = jnp.dot(x_ref[...], w_ref[...], preferred_element_type=jnp.float32)
    y_ref[...] = xw * dinv


_mm1 = pl.pallas_call(
    _mm1_body,
    out_shape=jax.ShapeDtypeStruct((NP, D), jnp.float32),
    grid=(GR,),
    in_specs=[
        pl.BlockSpec((BR, D), lambda i: (i, 0)),
        pl.BlockSpec((D, D), lambda i: (0, 0)),
        pl.BlockSpec((BR, NC), lambda i: (i, 0)),
    ],
    out_specs=pl.BlockSpec((BR, D), lambda i: (i, 0)),
)


def _mm2_body(y_ref, p_ref, dg_ref, b_ref, w_ref, o_ref):
    dinv = _dinv_of(dg_ref[...])
    agg = y_ref[...] + p_ref[0] + p_ref[1]
    h = jnp.maximum(agg * dinv + b_ref[...], 0.0)
    o_ref[...] = jnp.dot(h, w_ref[...], preferred_element_type=jnp.float32) * dinv


_mm2 = pl.pallas_call(
    _mm2_body,
    out_shape=jax.ShapeDtypeStruct((NP, D), jnp.float32),
    grid=(GR,),
    in_specs=[
        pl.BlockSpec((BR, D), lambda i: (i, 0)),
        pl.BlockSpec((NC, BR, D), lambda i: (0, i, 0)),
        pl.BlockSpec((BR, NC), lambda i: (i, 0)),
        pl.BlockSpec((1, D), lambda i: (0, 0)),
        pl.BlockSpec((D, D), lambda i: (0, 0)),
    ],
    out_specs=pl.BlockSpec((BR, D), lambda i: (i, 0)),
)


def _mm3_body(y_ref, p_ref, dg_ref, b_ref, o_ref):
    dinv = _dinv_of(dg_ref[...])
    agg = y_ref[...] + p_ref[0] + p_ref[1]
    o_ref[...] = agg * dinv + b_ref[...]


_mm3 = pl.pallas_call(
    _mm3_body,
    out_shape=jax.ShapeDtypeStruct((NP, D), jnp.float32),
    grid=(GR,),
    in_specs=[
        pl.BlockSpec((BR, D), lambda i: (i, 0)),
        pl.BlockSpec((NC, BR, D), lambda i: (0, i, 0)),
        pl.BlockSpec((BR, NC), lambda i: (i, 0)),
        pl.BlockSpec((1, D), lambda i: (0, 0)),
    ],
    out_specs=pl.BlockSpec((BR, D), lambda i: (i, 0)),
)


def kernel(edge_index, user_emb, item_emb, W1, b1, W2, b2):
    x = jnp.concatenate([user_emb, item_emb], axis=0)
    x = jnp.pad(x, ((0, NP - N), (0, 0)))
    src = edge_index[0].astype(jnp.int32)
    dst = edge_index[1].astype(jnp.int32)
    pad = jnp.full((EP - E,), NPA - 1, jnp.int32)
    src_r = jnp.concatenate([src, pad]).reshape(NW, NCH, C)
    src_r = jnp.pad(src_r, ((0, 0), (0, 2 * NBUF), (0, 0)))
    dst_flat = jnp.concatenate([dst, pad])
    dst_a = dst_flat.reshape(NW, NCH, C)
    dst_a = jnp.pad(dst_a, ((0, 0), (0, 2 * NBUF), (0, 0)))
    dst_d = dst_flat.reshape(NW, DNCH, DC)

    deg_t = _deg(dst_d).T
    y1 = _mm1(x, W1, deg_t)
    p = _agg(y1, src_r, dst_a)
    y2 = _mm2(y1, p, deg_t, b1.reshape(1, D), W2)
    q = _agg(y2, src_r, dst_a)
    out = _mm3(y2, q, deg_t, b2.reshape(1, D))
    return out[:N]


# speedup vs baseline: 1.0827x; 1.0827x over previous
import jax
import jax.numpy as jnp
from jax import lax
from jax.experimental import pallas as pl
from jax.experimental.pallas import tpu as pltpu
from jax.experimental.pallas import tpu_sc as plsc

N = 10000
D = 128
E = 320000

NC = 2
NS = 16
NW = NC * NS
C = 128
NCH = 80
ET = NCH * C
EP = NW * ET
NP = 10240
RPT = NP // NS
DC = 128
DNCH = ET // DC

_sc_mesh = plsc.VectorSubcoreMesh(core_axis_name="c", subcore_axis_name="s")


def _deg_body(dst_hbm, dp_hbm, dst_v, ones_v, zb_v, hist, ):
    cid = lax.axis_index("c")
    sid = lax.axis_index("s")
    wid = sid * NC + cid

    def fill_ones(i, _):
        ones_v[pl.ds(i * 16, 16)] = jnp.ones((16,), jnp.float32)
        return 0
    lax.fori_loop(0, DC // 16, fill_ones, 0)

    def fill_zero(i, _):
        zb_v[pl.ds(i * 16, 16)] = jnp.zeros((16,), jnp.float32)
        return 0
    lax.fori_loop(0, RPT // 16, fill_zero, 0)
    pltpu.sync_copy(zb_v, hist.at[pl.ds(sid * RPT, RPT)])
    plsc.subcore_barrier()

    pltpu.sync_copy(dst_hbm.at[wid], dst_v)

    def chunk(j, _):
        pltpu.sync_copy(ones_v, hist.at[dst_v.at[j]], add=True)
        return 0
    lax.fori_loop(0, DNCH, chunk, 0)

    plsc.subcore_barrier()
    pltpu.sync_copy(hist.at[pl.ds(sid * RPT, RPT)],
                    dp_hbm.at[cid, pl.ds(sid * RPT, RPT)])


_deg = pl.kernel(
    _deg_body,
    out_type=jax.ShapeDtypeStruct((NC, NP), jnp.float32),
    mesh=_sc_mesh,
    scratch_types=[
        pltpu.VMEM((DNCH, DC), jnp.int32),
        pltpu.VMEM((DC,), jnp.float32),
        pltpu.VMEM((RPT,), jnp.float32),
        pltpu.VMEM_SHARED((NP,), jnp.float32),
    ],
)


def _agg_body(y_hbm, src_hbm, dst_hbm, p_hbm, sbuf, dbuf, buf, acc):
    cid = lax.axis_index("c")
    sid = lax.axis_index("s")
    wid = sid * NC + cid

    def fill_zero(i, _):
        r = i // (D // 16)
        c0 = (i % (D // 16)) * 16
        buf[r, pl.ds(c0, 16)] = jnp.zeros((16,), jnp.float32)
        return 0
    lax.fori_loop(0, C * (D // 16), fill_zero, 0)
    for k in range(RPT // C):
        pltpu.sync_copy(buf, acc.at[pl.ds(sid * RPT + k * C, C)])
    plsc.subcore_barrier()

    pltpu.sync_copy(src_hbm.at[wid], sbuf)
    pltpu.sync_copy(dst_hbm.at[wid], dbuf)

    def chunk(j, _):
        pltpu.sync_copy(y_hbm.at[sbuf.at[j]], buf)
        pltpu.sync_copy(buf, acc.at[dbuf.at[j]], add=True)
        return 0
    lax.fori_loop(0, NCH, chunk, 0)

    plsc.subcore_barrier()
    pltpu.sync_copy(acc.at[pl.ds(sid * RPT, RPT)],
                    p_hbm.at[cid, pl.ds(sid * RPT, RPT)])


_agg = pl.kernel(
    _agg_body,
    out_type=jax.ShapeDtypeStruct((NC, NP, D), jnp.float32),
    mesh=_sc_mesh,
    scratch_types=[
        pltpu.VMEM((NCH, C), jnp.int32),
        pltpu.VMEM((NCH, C), jnp.int32),
        pltpu.VMEM((C, D), jnp.float32),
        pltpu.VMEM_SHARED((NP, D), jnp.float32),
    ],
)

BR = 256
GR = NP // BR


def _dinv_of(dg):
    return lax.rsqrt(dg[:, 0:1] + dg[:, 1:2] + 1.0)


def _mm1_body(x_ref, w_ref, dg_ref, y_ref):
    dinv = _dinv_of(dg_ref[...])
    xw = jnp.dot(x_ref[...], w_ref[...], preferred_element_type=jnp.float32)
    y_ref[...] = xw * dinv


_mm1 = pl.pallas_call(
    _mm1_body,
    out_shape=jax.ShapeDtypeStruct((NP, D), jnp.float32),
    grid=(GR,),
    in_specs=[
        pl.BlockSpec((BR, D), lambda i: (i, 0)),
        pl.BlockSpec((D, D), lambda i: (0, 0)),
        pl.BlockSpec((BR, NC), lambda i: (i, 0)),
    ],
    out_specs=pl.BlockSpec((BR, D), lambda i: (i, 0)),
)


def _mm2_body(y_ref, p_ref, dg_ref, b_ref, w_ref, o_ref):
    dinv = _dinv_of(dg_ref[...])
    agg = y_ref[...] + p_ref[0] + p_ref[1]
    h = jnp.maximum(agg * dinv + b_ref[...], 0.0)
    o_ref[...] = jnp.dot(h, w_ref[...], preferred_element_type=jnp.float32) * dinv


_mm2 = pl.pallas_call(
    _mm2_body,
    out_shape=jax.ShapeDtypeStruct((NP, D), jnp.float32),
    grid=(GR,),
    in_specs=[
        pl.BlockSpec((BR, D), lambda i: (i, 0)),
        pl.BlockSpec((NC, BR, D), lambda i: (0, i, 0)),
        pl.BlockSpec((BR, NC), lambda i: (i, 0)),
        pl.BlockSpec((1, D), lambda i: (0, 0)),
        pl.BlockSpec((D, D), lambda i: (0, 0)),
    ],
    out_specs=pl.BlockSpec((BR, D), lambda i: (i, 0)),
)


def _mm3_body(y_ref, p_ref, dg_ref, b_ref, o_ref):
    dinv = _dinv_of(dg_ref[...])
    agg = y_ref[...] + p_ref[0] + p_ref[1]
    o_ref[...] = agg * dinv + b_ref[...]


_mm3 = pl.pallas_call(
    _mm3_body,
    out_shape=jax.ShapeDtypeStruct((NP, D), jnp.float32),
    grid=(GR,),
    in_specs=[
        pl.BlockSpec((BR, D), lambda i: (i, 0)),
        pl.BlockSpec((NC, BR, D), lambda i: (0, i, 0)),
        pl.BlockSpec((BR, NC), lambda i: (i, 0)),
        pl.BlockSpec((1, D), lambda i: (0, 0)),
    ],
    out_specs=pl.BlockSpec((BR, D), lambda i: (i, 0)),
)


def kernel(edge_index, user_emb, item_emb, W1, b1, W2, b2):
    x = jnp.concatenate([user_emb, item_emb], axis=0)
    x = jnp.pad(x, ((0, NP - N), (0, 0)))
    src = edge_index[0].astype(jnp.int32)
    dst = edge_index[1].astype(jnp.int32)
    pad = jnp.full((EP - E,), NP - 1, jnp.int32)
    src_r = jnp.concatenate([src, pad]).reshape(NW, NCH, C)
    dst_flat = jnp.concatenate([dst, pad])
    dst_a = dst_flat.reshape(NW, NCH, C)
    dst_d = dst_flat.reshape(NW, DNCH, DC)

    deg_t = _deg(dst_d).T
    y1 = _mm1(x, W1, deg_t)
    p = _agg(y1, src_r, dst_a)
    y2 = _mm2(y1, p, deg_t, b1.reshape(1, D), W2)
    q = _agg(y2, src_r, dst_a)
    out = _mm3(y2, q, deg_t, b2.reshape(1, D))
    return out[:N]


# speedup vs baseline: 1.1646x; 1.0756x over previous
import jax
import jax.numpy as jnp
from jax import lax
from jax.experimental import pallas as pl
from jax.experimental.pallas import tpu as pltpu
from jax.experimental.pallas import tpu_sc as plsc

N = 10000
D = 128
E = 320000

NC = 2
NS = 16
NW = NC * NS
C = 128
NCH = 80
ET = NCH * C
EP = NW * ET
NP = 10240
RPT = NP // NS
DC = 128
DNCH = ET // DC

_sc_mesh = plsc.VectorSubcoreMesh(core_axis_name="c", subcore_axis_name="s")


def _deg_body(dst_hbm, dp_hbm, dst_v, ones_v, zb_v, hist, ):
    cid = lax.axis_index("c")
    sid = lax.axis_index("s")
    wid = sid * NC + cid

    def fill_ones(i, _):
        ones_v[pl.ds(i * 16, 16)] = jnp.ones((16,), jnp.float32)
        return 0
    lax.fori_loop(0, DC // 16, fill_ones, 0)

    def fill_zero(i, _):
        zb_v[pl.ds(i * 16, 16)] = jnp.zeros((16,), jnp.float32)
        return 0
    lax.fori_loop(0, RPT // 16, fill_zero, 0)
    pltpu.sync_copy(zb_v, hist.at[pl.ds(sid * RPT, RPT)])
    plsc.subcore_barrier()

    pltpu.sync_copy(dst_hbm.at[wid], dst_v)

    def chunk(j, _):
        pltpu.sync_copy(ones_v, hist.at[dst_v.at[j]], add=True)
        return 0
    lax.fori_loop(0, DNCH, chunk, 0)

    plsc.subcore_barrier()
    pltpu.sync_copy(hist.at[pl.ds(sid * RPT, RPT)],
                    dp_hbm.at[cid, pl.ds(sid * RPT, RPT)])


_deg = pl.kernel(
    _deg_body,
    out_type=jax.ShapeDtypeStruct((NC, NP), jnp.float32),
    mesh=_sc_mesh,
    scratch_types=[
        pltpu.VMEM((DNCH, DC), jnp.int32),
        pltpu.VMEM((DC,), jnp.float32),
        pltpu.VMEM((RPT,), jnp.float32),
        pltpu.VMEM_SHARED((NP,), jnp.float32),
    ],
)


def _agg_body(y_hbm, src_hbm, dst_hbm, p_hbm, sbuf, dbuf, buf, acc, *sems):
    cid = lax.axis_index("c")
    sid = lax.axis_index("s")
    wid = sid * NC + cid
    sem_i = sems[0:2]
    sem_d = sems[2:4]
    sem_g = sems[4:6]
    sem_s = sems[6:8]

    def fill_zero(i, _):
        r = i // (D // 16)
        c0 = (i % (D // 16)) * 16
        buf[0, r, pl.ds(c0, 16)] = jnp.zeros((16,), jnp.float32)
        return 0
    lax.fori_loop(0, C * (D // 16), fill_zero, 0)
    for k in range(RPT // C):
        pltpu.sync_copy(buf.at[0], acc.at[pl.ds(sid * RPT + k * C, C)])
    plsc.subcore_barrier()

    def sload(j, b):
        pltpu.async_copy(src_hbm.at[wid, j], sbuf.at[b], sem_i[b])

    def swait(j, b):
        pltpu.make_async_copy(src_hbm.at[wid, j], sbuf.at[b], sem_i[b]).wait()

    def dload(j, b):
        pltpu.async_copy(dst_hbm.at[wid, j], dbuf.at[b], sem_d[b])

    def dwait(j, b):
        pltpu.make_async_copy(dst_hbm.at[wid, j], dbuf.at[b], sem_d[b]).wait()

    def gstart(b):
        pltpu.async_copy(y_hbm.at[sbuf.at[b]], buf.at[b], sem_g[b])

    def gwait(b):
        pltpu.make_async_copy(y_hbm.at[sbuf.at[b]], buf.at[b], sem_g[b]).wait()

    def sstart(b):
        pltpu.async_copy(buf.at[b], acc.at[dbuf.at[b]], sem_s[b], add=True)

    def swait_s(b):
        pltpu.make_async_copy(buf.at[b], acc.at[dbuf.at[b]], sem_s[b]).wait()

    def steady(j, b, o, first=False, sl=True, dl=True, gs=True):
        gwait(b)
        if gs:
            swait(j + 1, o)
        dwait(j, b)
        sstart(b)
        if not first:
            swait_s(o)
        if gs:
            gstart(o)
        if dl:
            dload(j + 1, o)
        if sl:
            sload(j + 2, b)

    sload(0, 0)
    sload(1, 1)
    dload(0, 0)
    swait(0, 0)
    gstart(0)
    steady(0, 0, 1, first=True)

    def pair(i, _):
        steady(2 * i + 1, 1, 0)
        steady(2 * i + 2, 0, 1)
        return 0
    lax.fori_loop(0, (NCH - 4) // 2, pair, 0)
    steady(NCH - 3, 1, 0)
    steady(NCH - 2, 0, 1, sl=False)
    steady(NCH - 1, 1, 0, sl=False, dl=False, gs=False)
    swait_s(1)

    plsc.subcore_barrier()
    pltpu.sync_copy(acc.at[pl.ds(sid * RPT, RPT)],
                    p_hbm.at[cid, pl.ds(sid * RPT, RPT)])


_agg = pl.kernel(
    _agg_body,
    out_type=jax.ShapeDtypeStruct((NC, NP, D), jnp.float32),
    mesh=_sc_mesh,
    scratch_types=[
        pltpu.VMEM((2, C), jnp.int32),
        pltpu.VMEM((2, C), jnp.int32),
        pltpu.VMEM((2, C, D), jnp.float32),
        pltpu.VMEM_SHARED((NP, D), jnp.float32),
    ] + [pltpu.SemaphoreType.DMA] * 8,
)

BR = 256
GR = NP // BR


def _dinv_of(dg):
    return lax.rsqrt(dg[:, 0:1] + dg[:, 1:2] + 1.0)


def _mm1_body(x_ref, w_ref, dg_ref, y_ref):
    dinv = _dinv_of(dg_ref[...])
    xw = jnp.dot(x_ref[...], w_ref[...], preferred_element_type=jnp.float32)
    y_ref[...] = xw * dinv


_mm1 = pl.pallas_call(
    _mm1_body,
    out_shape=jax.ShapeDtypeStruct((NP, D), jnp.float32),
    grid=(GR,),
    in_specs=[
        pl.BlockSpec((BR, D), lambda i: (i, 0)),
        pl.BlockSpec((D, D), lambda i: (0, 0)),
        pl.BlockSpec((BR, NC), lambda i: (i, 0)),
    ],
    out_specs=pl.BlockSpec((BR, D), lambda i: (i, 0)),
)


def _mm2_body(y_ref, p_ref, dg_ref, b_ref, w_ref, o_ref):
    dinv = _dinv_of(dg_ref[...])
    agg = y_ref[...] + p_ref[0] + p_ref[1]
    h = jnp.maximum(agg * dinv + b_ref[...], 0.0)
    o_ref[...] = jnp.dot(h, w_ref[...], preferred_element_type=jnp.float32) * dinv


_mm2 = pl.pallas_call(
    _mm2_body,
    out_shape=jax.ShapeDtypeStruct((NP, D), jnp.float32),
    grid=(GR,),
    in_specs=[
        pl.BlockSpec((BR, D), lambda i: (i, 0)),
        pl.BlockSpec((NC, BR, D), lambda i: (0, i, 0)),
        pl.BlockSpec((BR, NC), lambda i: (i, 0)),
        pl.BlockSpec((1, D), lambda i: (0, 0)),
        pl.BlockSpec((D, D), lambda i: (0, 0)),
    ],
    out_specs=pl.BlockSpec((BR, D), lambda i: (i, 0)),
)


def _mm3_body(y_ref, p_ref, dg_ref, b_ref, o_ref):
    dinv = _dinv_of(dg_ref[...])
    agg = y_ref[...] + p_ref[0] + p_ref[1]
    o_ref[...] = agg * dinv + b_ref[...]


_mm3 = pl.pallas_call(
    _mm3_body,
    out_shape=jax.ShapeDtypeStruct((NP, D), jnp.float32),
    grid=(GR,),
    in_specs=[
        pl.BlockSpec((BR, D), lambda i: (i, 0)),
        pl.BlockSpec((NC, BR, D), lambda i: (0, i, 0)),
        pl.BlockSpec((BR, NC), lambda i: (i, 0)),
        pl.BlockSpec((1, D), lambda i: (0, 0)),
    ],
    out_specs=pl.BlockSpec((BR, D), lambda i: (i, 0)),
)


def kernel(edge_index, user_emb, item_emb, W1, b1, W2, b2):
    x = jnp.concatenate([user_emb, item_emb], axis=0)
    x = jnp.pad(x, ((0, NP - N), (0, 0)))
    src = edge_index[0].astype(jnp.int32)
    dst = edge_index[1].astype(jnp.int32)
    pad = jnp.full((EP - E,), NP - 1, jnp.int32)
    src_r = jnp.concatenate([src, pad]).reshape(NW, NCH, C)
    dst_flat = jnp.concatenate([dst, pad])
    dst_a = dst_flat.reshape(NW, NCH, C)
    dst_d = dst_flat.reshape(NW, DNCH, DC)

    deg_t = _deg(dst_d).T
    y1 = _mm1(x, W1, deg_t)
    p = _agg(y1, src_r, dst_a)
    y2 = _mm2(y1, p, deg_t, b1.reshape(1, D), W2)
    q = _agg(y2, src_r, dst_a)
    out = _mm3(y2, q, deg_t, b2.reshape(1, D))
    return out[:N]
